# Initial kernel scaffold; baseline (speedup 1.0000x reference)
import jax
import jax.numpy as jnp
from jax.experimental import pallas as pl


def kernel(x, W_weights, W_emb):
    raise NotImplementedError("write your pallas kernel here")


# speedup vs baseline: 20.9941x; 20.9941x over previous
import functools

import jax
import jax.numpy as jnp
from jax import lax
from jax.experimental import pallas as pl
from jax.experimental.pallas import tpu as pltpu
from jax.experimental.pallas import tpu_sc as plsc

NUM_H = 4
KV = 513
NE = 257
DIM = 64
ROWS = NUM_H * KV
NC, NS, L = 2, 16, 16
NW = NC * NS

C = 512
JB = C // 128


def _t2_body(ww_ref, we_ref, t2_ref):
    r = lax.broadcasted_iota(jnp.int32, (ROWS, NE), 0)
    c = lax.broadcasted_iota(jnp.int32, (ROWS, NE), 1)
    e = lax.rem(r, KV) // 2
    onehot = jnp.where(c == e, 1.0, 0.0)
    emb = jnp.dot(onehot, we_ref[...], preferred_element_type=jnp.float32)
    t2_ref[...] = emb * ww_ref[...]


def _build_t2(W_weights, W_emb):
    return pl.pallas_call(
        _t2_body,
        out_shape=jax.ShapeDtypeStruct((ROWS, DIM), jnp.float32),
    )(W_weights, W_emb)


@functools.lru_cache(maxsize=None)
def _make_sc_lookup(N):
    SPW = N // NW
    NCHUNK = SPW // C

    mesh = plsc.VectorSubcoreMesh(core_axis_name="c", subcore_axis_name="s")

    @functools.partial(
        pl.kernel,
        out_type=jax.ShapeDtypeStruct((N, DIM), jnp.float32),
        mesh=mesh,
        scratch_types=[
            pltpu.VMEM((C * NUM_H,), jnp.int32),
            pltpu.VMEM((NUM_H, JB, 128), jnp.int32),
            pltpu.VMEM((C, DIM), jnp.float32),
            pltpu.SemaphoreType.DMA,
        ],
        compiler_params=pltpu.CompilerParams(
            needs_layout_passes=False, use_tc_tiling_on_sc=False
        ),
    )
    def sc_lookup(x_hbm, t2_hbm, out_hbm, x_v, widx_v, out_v, sem):
        cid = lax.axis_index("c")
        sid = lax.axis_index("s")
        wid = sid * NC + cid
        base = wid * SPW
        iota = lax.iota(jnp.int32, 16)

        def chunk(ci, carry):
            s0 = base + ci * C
            pltpu.sync_copy(x_hbm.at[pl.ds(s0 * NUM_H, C * NUM_H)], x_v)

            for h in range(NUM_H):
                for j in range(JB):
                    def idx_body(t, _, h=h, j=j):
                        pos = iota * NUM_H + (t * 64 + (j * 512 + h))
                        v = plsc.load_gather(x_v, [pos])
                        widx_v[h, j, pl.ds(t * 16, 16)] = v + jnp.int32(KV * h)
                        return _
                    lax.fori_loop(0, 8, idx_body, 0, unroll=True)

            first = [
                pltpu.async_copy(
                    t2_hbm.at[widx_v.at[0, j]],
                    out_v.at[pl.ds(j * 128, 128)],
                    sem,
                )
                for j in range(JB)
            ]
            for d in first:
                d.wait()
            adds = [
                pltpu.async_copy(
                    t2_hbm.at[widx_v.at[h, j]],
                    out_v.at[pl.ds(j * 128, 128)],
                    sem,
                    add=True,
                )
                for h in range(1, NUM_H)
                for j in range(JB)
            ]
            for d in adds:
                d.wait()

            pltpu.sync_copy(out_v, out_hbm.at[pl.ds(s0, C)])
            return carry

        lax.fori_loop(0, NCHUNK, chunk, 0)

    return sc_lookup


def kernel(x, W_weights, W_emb):
    B, S, H = x.shape
    N = B * S
    t2 = _build_t2(W_weights, W_emb)
    xf = x.reshape(-1).astype(jnp.int32)
    out = _make_sc_lookup(N)(xf, t2)
    return out.reshape(B, S, DIM)
